# baseline (speedup 1.0000x reference)
import functools

import jax
import jax.numpy as jnp
from jax import lax
from jax.experimental import pallas as pl
from jax.experimental.pallas import tpu as pltpu
from jax.experimental.pallas import tpu_sc as plsc

EMBED_DIM = 64
NUM_EMB = 1024
COMMIT = 0.25
MOMENTUM = 0.9

ROWS = 32 * 1024
BLK = 2048
NBLK = ROWS // BLK

NC = 2
NS = 16
NW = NC * NS
BW = ROWS // NW
CH = 128
NCH = BW // CH


def _main_body(x_ref, e_ref, qst_ref, idx_ref, idx2_ref, loss_ref, cnt_ref,
               cnt_acc, loss_acc):
    i = pl.program_id(0)
    xb = x_ref[...]
    e = e_ref[...]
    x2 = jnp.sum(xb * xb, axis=1, keepdims=True)
    e2 = jnp.sum(e * e, axis=0, keepdims=True)
    s = lax.dot_general(xb, e, (((1,), (0,)), ((), ())),
                        preferred_element_type=jnp.float32)
    dist = (x2 + e2) - 2.0 * s
    m = jnp.min(dist, axis=1, keepdims=True)
    codes = lax.broadcasted_iota(jnp.int32, (BLK, NUM_EMB), 1)
    cand = jnp.where(dist == m, codes, NUM_EMB)
    idx = jnp.min(cand, axis=1)
    idx_ref[...] = idx
    idx2_ref[...] = idx.reshape(BLK // 128, 128)

    enc = (codes == idx[:, None]).astype(jnp.float32)
    q = lax.dot_general(enc, e, (((1,), (1,)), ((), ())),
                        preferred_element_type=jnp.float32)
    qst_ref[...] = xb + (q - xb)
    loss_part = jnp.sum((q - xb) ** 2).reshape(1, 1)
    ones = jnp.ones((1, BLK), jnp.float32)
    cnt_part = lax.dot_general(ones, enc, (((1,), (0,)), ((), ())),
                               preferred_element_type=jnp.float32)

    @pl.when(i == 0)
    def _init():
        cnt_acc[...] = cnt_part
        loss_acc[...] = loss_part

    @pl.when(i > 0)
    def _acc():
        cnt_acc[...] += cnt_part
        loss_acc[...] += loss_part

    @pl.when(i == NBLK - 1)
    def _fin():
        cnt_ref[...] = cnt_acc[...]
        loss_ref[...] = COMMIT * loss_acc[...] * (1.0 / (ROWS * EMBED_DIM))


def _stage_main(xf, e):
    scalar = pl.BlockSpec((1, 1), lambda i: (0, 0))
    return pl.pallas_call(
        _main_body,
        grid=(NBLK,),
        in_specs=[
            pl.BlockSpec((BLK, EMBED_DIM), lambda i: (i, 0)),
            pl.BlockSpec((EMBED_DIM, NUM_EMB), lambda i: (0, 0)),
        ],
        out_specs=(
            pl.BlockSpec((BLK, EMBED_DIM), lambda i: (i, 0)),
            pl.BlockSpec((BLK,), lambda i: (i,)),
            pl.BlockSpec((BLK // 128, 128), lambda i: (i, 0)),
            scalar,
            pl.BlockSpec((1, NUM_EMB), lambda i: (0, 0)),
        ),
        out_shape=(
            jax.ShapeDtypeStruct((ROWS, EMBED_DIM), jnp.float32),
            jax.ShapeDtypeStruct((ROWS,), jnp.int32),
            jax.ShapeDtypeStruct((ROWS // 128, 128), jnp.int32),
            jax.ShapeDtypeStruct((1, 1), jnp.float32),
            jax.ShapeDtypeStruct((1, NUM_EMB), jnp.float32),
        ),
        scratch_shapes=[
            pltpu.VMEM((1, NUM_EMB), jnp.float32),
            pltpu.VMEM((1, 1), jnp.float32),
        ],
    )(xf, e)


def _sc_body(idx_ref, x_ref, zun_ref, un_ref, idx_v, buf_v, sh_un, sem):
    c = lax.axis_index("c")
    s = lax.axis_index("s")
    w = s * NC + c
    base = w * BW

    @pl.when(s == 0)
    def _zero():
        pltpu.sync_copy(zun_ref, sh_un)

    pltpu.sync_copy(idx_ref.at[pl.ds(w * NCH, NCH)], idx_v)
    pltpu.sync_copy(x_ref.at[pl.ds(base, BW)], buf_v)

    plsc.subcore_barrier()

    for j in range(NCH):
        pltpu.sync_copy(buf_v.at[pl.ds(j * CH, CH)],
                        sh_un.at[idx_v.at[j]], add=True)

    plsc.subcore_barrier()

    @pl.when(s == 0)
    def _drain():
        pltpu.sync_copy(sh_un, un_ref.at[c])


def _sc_scatter(idx2, xf, zun):
    mesh = plsc.VectorSubcoreMesh(core_axis_name="c", subcore_axis_name="s")
    f = functools.partial(
        pl.kernel,
        out_type=jax.ShapeDtypeStruct((NC, NUM_EMB, EMBED_DIM), jnp.float32),
        mesh=mesh,
        scratch_types=[
            pltpu.VMEM((NCH, CH), jnp.int32),
            pltpu.VMEM((BW, EMBED_DIM), jnp.float32),
            pltpu.VMEM_SHARED((NUM_EMB, EMBED_DIM), jnp.float32),
            pltpu.SemaphoreType.DMA,
        ],
        compiler_params=pltpu.CompilerParams(use_tc_tiling_on_sc=False),
    )(_sc_body)
    return f(idx2, xf, zun)


def _fin_body(cnt_ref, unp_ref, cs_ref, un_ref,
              ppl_ref, newe_ref, newcs_ref, newun_ref):
    counts = cnt_ref[...].reshape(NUM_EMB)
    new_cs = (1.0 - MOMENTUM) * counts + MOMENTUM * cs_ref[...]
    n = jnp.sum(new_cs)
    stable_cs = (new_cs + 1e-20) / (n + NUM_EMB * 1e-20) * n
    unp = unp_ref[...]
    un_t = jnp.transpose(unp[0] + unp[1])
    new_un = (1.0 - MOMENTUM) * un_t + MOMENTUM * un_ref[...]
    newcs_ref[...] = new_cs
    newun_ref[...] = new_un
    newe_ref[...] = new_un / stable_cs[None, :]
    probs = counts * (1.0 / ROWS)
    ppl_ref[...] = jnp.exp(
        -jnp.sum(probs * jnp.log(probs + 1e-20))).reshape(1, 1)


def _finalize(cnt, un_p, cluster_size, unnormalized):
    return pl.pallas_call(
        _fin_body,
        out_shape=(
            jax.ShapeDtypeStruct((1, 1), jnp.float32),
            jax.ShapeDtypeStruct((EMBED_DIM, NUM_EMB), jnp.float32),
            jax.ShapeDtypeStruct((NUM_EMB,), jnp.float32),
            jax.ShapeDtypeStruct((EMBED_DIM, NUM_EMB), jnp.float32),
        ),
    )(cnt, un_p, cluster_size, unnormalized)


def kernel(x, embeddings, cluster_size, unnormalized):
    input_shape = x.shape[:-1]
    xf = x.reshape((-1, EMBED_DIM))

    qst, idx, idx2, loss, cnt = _stage_main(xf, embeddings)

    zun = jnp.zeros((NUM_EMB, EMBED_DIM), jnp.float32)
    un_p = _sc_scatter(idx2, xf, zun)

    ppl, new_e, new_cs, new_un = _finalize(cnt, un_p, cluster_size,
                                           unnormalized)

    return (qst.reshape((*input_shape, EMBED_DIM)), loss.reshape(()),
            ppl.reshape(()), idx, new_e, new_cs, new_un)


# speedup vs baseline: 1.0076x; 1.0076x over previous
import functools

import jax
import jax.numpy as jnp
from jax import lax
from jax.experimental import pallas as pl
from jax.experimental.pallas import tpu as pltpu
from jax.experimental.pallas import tpu_sc as plsc

EMBED_DIM = 64
NUM_EMB = 1024
COMMIT = 0.25
MOMENTUM = 0.9

ROWS = 32 * 1024
BLK = 4096
NBLK = ROWS // BLK

NC = 2
NS = 16
NW = NC * NS
BW = ROWS // NW
CH = 128
NCH = BW // CH


def _main_body(x_ref, e_ref, qst_ref, idx_ref, idx2_ref, loss_ref,
               cnt_ref, cnt_acc, loss_acc):
    i = pl.program_id(0)
    xb = x_ref[...]
    e = e_ref[...]
    x2 = jnp.sum(xb * xb, axis=1, keepdims=True)
    e2 = jnp.sum(e * e, axis=0, keepdims=True)
    s = lax.dot_general(xb, e, (((1,), (0,)), ((), ())),
                        preferred_element_type=jnp.float32)
    dist = (x2 + e2) - 2.0 * s
    m = jnp.min(dist, axis=1, keepdims=True)
    codes = lax.broadcasted_iota(jnp.int32, (BLK, NUM_EMB), 1)
    cand = jnp.where(dist == m, codes, NUM_EMB)
    idx = jnp.min(cand, axis=1)
    idx_ref[...] = idx
    idx2_ref[...] = idx.reshape(BLK // 128, 128)

    enc = (codes == idx[:, None]).astype(jnp.float32)
    q = lax.dot_general(enc, e, (((1,), (1,)), ((), ())),
                        preferred_element_type=jnp.float32)
    qst_ref[...] = xb + (q - xb)
    loss_part = jnp.sum((q - xb) ** 2).reshape(1, 1)
    ones = jnp.ones((1, BLK), jnp.float32)
    cnt_part = lax.dot_general(ones, enc, (((1,), (0,)), ((), ())),
                               preferred_element_type=jnp.float32)

    @pl.when(i == 0)
    def _init():
        cnt_acc[...] = cnt_part
        loss_acc[...] = loss_part

    @pl.when(i > 0)
    def _acc():
        cnt_acc[...] += cnt_part
        loss_acc[...] += loss_part

    @pl.when(i == NBLK - 1)
    def _fin():
        cnt_ref[...] = cnt_acc[...]
        loss_ref[...] = COMMIT * loss_acc[...] * (1.0 / (ROWS * EMBED_DIM))


def _stage_main(xf, e):
    scalar = pl.BlockSpec((1, 1), lambda i: (0, 0))
    return pl.pallas_call(
        _main_body,
        grid=(NBLK,),
        in_specs=[
            pl.BlockSpec((BLK, EMBED_DIM), lambda i: (i, 0)),
            pl.BlockSpec((EMBED_DIM, NUM_EMB), lambda i: (0, 0)),
        ],
        out_specs=(
            pl.BlockSpec((BLK, EMBED_DIM), lambda i: (i, 0)),
            pl.BlockSpec((BLK,), lambda i: (i,)),
            pl.BlockSpec((BLK // 128, 128), lambda i: (i, 0)),
            scalar,
            pl.BlockSpec((1, NUM_EMB), lambda i: (0, 0)),
        ),
        out_shape=(
            jax.ShapeDtypeStruct((ROWS, EMBED_DIM), jnp.float32),
            jax.ShapeDtypeStruct((ROWS,), jnp.int32),
            jax.ShapeDtypeStruct((ROWS // 128, 128), jnp.int32),
            jax.ShapeDtypeStruct((1, 1), jnp.float32),
            jax.ShapeDtypeStruct((1, NUM_EMB), jnp.float32),
        ),
        scratch_shapes=[
            pltpu.VMEM((1, NUM_EMB), jnp.float32),
            pltpu.VMEM((1, 1), jnp.float32),
        ],
    )(xf, e)


def _sc_body(idx_ref, x_ref, zun_ref, un_ref, idx_v, buf_v, sh_un, sem):
    c = lax.axis_index("c")
    s = lax.axis_index("s")
    w = s * NC + c
    base = w * BW

    @pl.when(s == 0)
    def _zero():
        pltpu.sync_copy(zun_ref, sh_un)

    pltpu.sync_copy(idx_ref.at[pl.ds(w * NCH, NCH)], idx_v)
    pltpu.sync_copy(x_ref.at[pl.ds(base, BW)], buf_v)

    plsc.subcore_barrier()

    for j in range(NCH):
        pltpu.sync_copy(buf_v.at[pl.ds(j * CH, CH)],
                        sh_un.at[idx_v.at[j]], add=True)

    plsc.subcore_barrier()

    @pl.when(s == 0)
    def _drain():
        pltpu.sync_copy(sh_un, un_ref.at[c])


def _sc_scatter(idx2, xf, zun):
    mesh = plsc.VectorSubcoreMesh(core_axis_name="c", subcore_axis_name="s")
    f = functools.partial(
        pl.kernel,
        out_type=jax.ShapeDtypeStruct((NC, NUM_EMB, EMBED_DIM), jnp.float32),
        mesh=mesh,
        scratch_types=[
            pltpu.VMEM((NCH, CH), jnp.int32),
            pltpu.VMEM((BW, EMBED_DIM), jnp.float32),
            pltpu.VMEM_SHARED((NUM_EMB, EMBED_DIM), jnp.float32),
            pltpu.SemaphoreType.DMA,
        ],
        compiler_params=pltpu.CompilerParams(use_tc_tiling_on_sc=False),
    )(_sc_body)
    return f(idx2, xf, zun)


def _fin_body(cnt_ref, unp_ref, cs_ref, un_ref,
              ppl_ref, newe_ref, newcs_ref, newun_ref):
    counts = cnt_ref[...].reshape(NUM_EMB)
    new_cs = (1.0 - MOMENTUM) * counts + MOMENTUM * cs_ref[...]
    n = jnp.sum(new_cs)
    stable_cs = (new_cs + 1e-20) / (n + NUM_EMB * 1e-20) * n
    unp = unp_ref[...]
    un_t = jnp.transpose(unp[0] + unp[1])
    new_un = (1.0 - MOMENTUM) * un_t + MOMENTUM * un_ref[...]
    newcs_ref[...] = new_cs
    newun_ref[...] = new_un
    newe_ref[...] = new_un / stable_cs[None, :]
    probs = counts * (1.0 / ROWS)
    ppl_ref[...] = jnp.exp(
        -jnp.sum(probs * jnp.log(probs + 1e-20))).reshape(1, 1)


def _finalize(cnt, un_p, cluster_size, unnormalized):
    return pl.pallas_call(
        _fin_body,
        out_shape=(
            jax.ShapeDtypeStruct((1, 1), jnp.float32),
            jax.ShapeDtypeStruct((EMBED_DIM, NUM_EMB), jnp.float32),
            jax.ShapeDtypeStruct((NUM_EMB,), jnp.float32),
            jax.ShapeDtypeStruct((EMBED_DIM, NUM_EMB), jnp.float32),
        ),
    )(cnt, un_p, cluster_size, unnormalized)


def kernel(x, embeddings, cluster_size, unnormalized):
    input_shape = x.shape[:-1]
    xf = x.reshape((-1, EMBED_DIM))

    qst, idx, idx2, loss, cnt = _stage_main(xf, embeddings)

    zun = jnp.zeros((NUM_EMB, EMBED_DIM), jnp.float32)
    un_p = _sc_scatter(idx2, xf, zun)

    ppl, new_e, new_cs, new_un = _finalize(cnt, un_p, cluster_size,
                                           unnormalized)

    return (qst.reshape((*input_shape, EMBED_DIM)), loss.reshape(()),
            ppl.reshape(()), idx, new_e, new_cs, new_un)
